# baseline (speedup 1.0000x reference)
import functools

import jax
import jax.numpy as jnp
from jax import lax
from jax.experimental import pallas as pl
from jax.experimental.pallas import tpu as pltpu
from jax.experimental.pallas import tpu_sc as plsc

NUM_CORES = 2
NUM_SUBCORES = 16
NUM_WORKERS = NUM_CORES * NUM_SUBCORES
NBUF = 8
SLACK = 2


def _make_emb_kernel(batch: int, seq: int, seq_pad: int, vocab: int, d: int):
  per_w = batch // NUM_WORKERS
  assert (per_w - NBUF) % NBUF == 0 and per_w > NBUF + SLACK
  n_groups = (per_w - NBUF) // NBUF
  mesh = plsc.VectorSubcoreMesh(core_axis_name="c", subcore_axis_name="s")

  @functools.partial(
      pl.kernel,
      mesh=mesh,
      out_type=jax.ShapeDtypeStruct((batch, seq_pad, d), jnp.float32),
      scratch_types=[
          pltpu.VMEM((per_w, seq), jnp.int32),
          pltpu.VMEM((NBUF, seq_pad, d), jnp.float32),
      ] + [pltpu.SemaphoreType.DMA] * (2 * NBUF),
  )
  def emb(idx_hbm, tab_hbm, out_hbm, idx_v, rows_v, *sems):
    gsems, ssems = sems[:NBUF], sems[NBUF:]
    wid = lax.axis_index("s") * NUM_CORES + lax.axis_index("c")
    base = wid * per_w
    pltpu.sync_copy(idx_hbm.at[wid], idx_v)

    def gather_start(k, b):
      pltpu.async_copy(
          tab_hbm.at[idx_v.at[k]], rows_v.at[b, pl.ds(0, seq)], gsems[b])

    def gather_wait(k, b):
      pltpu.make_async_copy(
          tab_hbm.at[idx_v.at[k]], rows_v.at[b, pl.ds(0, seq)], gsems[b]).wait()

    def store_start(k, b):
      pltpu.async_copy(rows_v.at[b], out_hbm.at[base + k], ssems[b])

    def store_wait(k, b):
      pltpu.make_async_copy(
          rows_v.at[b], out_hbm.at[base + k], ssems[b]).wait()

    for b in range(NBUF):
      gather_start(b, b)
    for k in range(SLACK):
      gather_wait(k, k)
      store_start(k, k)

    def group(g):
      for i in range(NBUF):
        k = SLACK + g * NBUF + i
        b = (SLACK + i) % NBUF
        br = i
        store_wait(k - SLACK, br)
        gather_start(k - SLACK + NBUF, br)
        gather_wait(k, b)
        store_start(k, b)

    pl.loop(0, n_groups)(group)

    for k in range(per_w - NBUF + SLACK, per_w):
      store_wait(k - SLACK, (k - SLACK) % NBUF)
      gather_wait(k, k % NBUF)
      store_start(k, k % NBUF)
    for k in range(per_w - SLACK, per_w):
      store_wait(k, k % NBUF)

  return emb


def kernel(word_indices, embedding_table):
  batch, seq = word_indices.shape
  vocab, d = embedding_table.shape
  seq_pad = (seq + 7) // 8 * 8
  idx3 = word_indices.astype(jnp.int32).reshape(
      NUM_WORKERS, batch // NUM_WORKERS, seq)
  emb = _make_emb_kernel(batch, seq, seq_pad, vocab, d)
  out_padded = emb(idx3, embedding_table)
  return out_padded[:, :seq, :]


# speedup vs baseline: 1.1743x; 1.1743x over previous
import functools

import jax
import jax.numpy as jnp
from jax import lax
from jax.experimental import pallas as pl
from jax.experimental.pallas import tpu as pltpu
from jax.experimental.pallas import tpu_sc as plsc

NUM_CORES = 2
NUM_SUBCORES = 16
NUM_WORKERS = NUM_CORES * NUM_SUBCORES
NBUF = 8
SLACK = 2


def _make_emb_kernel(batch: int, seq: int, vocab: int, d: int):
  per_w = batch // NUM_WORKERS
  assert (per_w - NBUF) % NBUF == 0 and per_w > NBUF + SLACK
  n_groups = (per_w - NBUF) // NBUF
  mesh = plsc.VectorSubcoreMesh(core_axis_name="c", subcore_axis_name="s")

  @functools.partial(
      pl.kernel,
      mesh=mesh,
      out_type=jax.ShapeDtypeStruct((batch, seq, d), jnp.float32),
      compiler_params=pltpu.CompilerParams(needs_layout_passes=True),
      scratch_types=[
          pltpu.VMEM((per_w, seq), jnp.int32),
          pltpu.VMEM((NBUF, seq, d), jnp.float32),
      ] + [pltpu.SemaphoreType.DMA] * (2 * NBUF),
  )
  def emb(idx_hbm, tab_hbm, out_hbm, idx_v, rows_v, *sems):
    gsems, ssems = sems[:NBUF], sems[NBUF:]
    wid = lax.axis_index("s") * NUM_CORES + lax.axis_index("c")
    base = wid * per_w
    pltpu.sync_copy(idx_hbm.at[wid], idx_v)

    def gather_start(k, b):
      pltpu.async_copy(tab_hbm.at[idx_v.at[k]], rows_v.at[b], gsems[b])

    def gather_wait(k, b):
      pltpu.make_async_copy(
          tab_hbm.at[idx_v.at[k]], rows_v.at[b], gsems[b]).wait()

    def store_start(k, b):
      pltpu.async_copy(rows_v.at[b], out_hbm.at[base + k], ssems[b])

    def store_wait(k, b):
      pltpu.make_async_copy(
          rows_v.at[b], out_hbm.at[base + k], ssems[b]).wait()

    for b in range(NBUF):
      gather_start(b, b)
    for k in range(SLACK):
      gather_wait(k, k)
      store_start(k, k)

    def group(g):
      for i in range(NBUF):
        k = SLACK + g * NBUF + i
        b = (SLACK + i) % NBUF
        br = i
        store_wait(k - SLACK, br)
        gather_start(k - SLACK + NBUF, br)
        gather_wait(k, b)
        store_start(k, b)

    pl.loop(0, n_groups)(group)

    for k in range(per_w - NBUF + SLACK, per_w):
      store_wait(k - SLACK, (k - SLACK) % NBUF)
      gather_wait(k, k % NBUF)
      store_start(k, k % NBUF)
    for k in range(per_w - SLACK, per_w):
      store_wait(k, k % NBUF)

  return emb


def kernel(word_indices, embedding_table):
  batch, seq = word_indices.shape
  vocab, d = embedding_table.shape
  idx3 = word_indices.astype(jnp.int32).reshape(
      NUM_WORKERS, batch // NUM_WORKERS, seq)
  emb = _make_emb_kernel(batch, seq, vocab, d)
  return emb(idx3, embedding_table)
